# baseline (device time: 18003 ns/iter reference)
import os

import jax
import jax.numpy as jnp
from jax import lax
from jax.experimental import pallas as pl
from jax.experimental.pallas import tpu as pltpu

ABLATE = os.environ.get("ABLATE", "")

N_DEV = 4
WINDOW = 128
CDT = jnp.bfloat16


def kernel(x, Wq, K_ext, V_ext, Wo):
    B, Sq, D_in = x.shape
    _, Skv, Hl, Dh = K_ext.shape
    D_out = Wo.shape[1]
    HD = Hl * Dh
    HALF = HD // 2

    pos = lax.axis_index("i")
    x_b = x.astype(CDT)
    wq_b = lax.dynamic_slice(Wq, (0, pos * HD), (D_in, HD)).astype(CDT)

    def body(x_ref, wq_ref, k_ref, v_ref, wo_ref, out_ref,
             kvv, wov, comm_ref, kv_sems, wo_sem,
             send_sems, recv_sems):
        my_pos = lax.axis_index("i")
        right = lax.rem(my_pos + 1, N_DEV)
        left = lax.rem(my_pos + N_DEV - 1, N_DEV)

        cp_kv = {}
        for b in range(B):
            for h in range(Hl):
                cp_kv[(0, b, h)] = pltpu.make_async_copy(
                    k_ref.at[b, :, h, :], kvv.at[0, b, h],
                    kv_sems.at[0 * B * Hl + b * Hl + h])
                cp_kv[(1, b, h)] = pltpu.make_async_copy(
                    v_ref.at[b, :, h, :], kvv.at[1, b, h],
                    kv_sems.at[1 * B * Hl + b * Hl + h])
        for h in range(Hl):
            for b in range(B):
                cp_kv[(0, b, h)].start()
                cp_kv[(1, b, h)].start()
        cp_wo = pltpu.make_async_copy(wo_ref, wov, wo_sem)
        cp_wo.start()

        barrier_sem = pltpu.get_barrier_semaphore()
        for nbr in (left, right):
            pl.semaphore_signal(
                barrier_sem, inc=1,
                device_id=(nbr,), device_id_type=pl.DeviceIdType.MESH,
            )

        rows = lax.broadcasted_iota(jnp.int32, (Sq, Skv), 0)
        cols = lax.broadcasted_iota(jnp.int32, (Sq, Skv), 1)
        mask = jnp.abs(rows - cols) <= WINDOW

        x_all = x_ref[...].reshape(B * Sq, D_in)
        q_all = jnp.dot(x_all, wq_ref[...],
                        preferred_element_type=jnp.float32) * 0.125

        def attend(b, h):
            qh = q_all[b * Sq:(b + 1) * Sq, h * Dh:(h + 1) * Dh].astype(CDT)
            if ABLATE == "noattn":
                comm_ref[0, b, :, h * Dh:(h + 1) * Dh] = qh
                return
            cp_kv[(0, b, h)].wait()
            cp_kv[(1, b, h)].wait()
            kh = kvv[0, b, h].astype(CDT)
            vh = kvv[1, b, h].astype(CDT)
            s = lax.dot_general(
                qh, kh, (((1,), (1,)), ((), ())),
                preferred_element_type=jnp.float32,
            )
            w = jnp.exp(jnp.where(mask, s, -1e9))
            recip = 1.0 / jnp.sum(w, axis=1, keepdims=True)
            ctx = jnp.dot(w.astype(CDT), vh,
                          preferred_element_type=jnp.float32)
            comm_ref[0, b, :, h * Dh:(h + 1) * Dh] = (ctx * recip).astype(CDT)

        SLOT_ORIGIN_DELTA = {0: 0, 1: N_DEV - 1, 2: 1, 3: 2}

        def contrib(slot, bs=None, init=False):
            if ABLATE == "nocontrib" and slot > 0:
                return
            origin = lax.rem(my_pos + SLOT_ORIGIN_DELTA[slot], N_DEV)
            wo_block = wov[pl.ds(origin * HD, HD), :].astype(CDT)
            if bs is None:
                delta = jnp.dot(
                    comm_ref[slot].reshape(B * Sq, HD), wo_block,
                    preferred_element_type=jnp.float32,
                ).reshape(B, Sq, D_out)
                out_ref[...] = delta if init else out_ref[...] + delta
            else:
                for b in bs:
                    delta = jnp.dot(
                        comm_ref[slot, b], wo_block,
                        preferred_element_type=jnp.float32,
                    )
                    out_ref[b] = delta if init else out_ref[b] + delta

        def copy(src_at, dst_at, sem_idx, target):
            return pltpu.make_async_remote_copy(
                src_ref=src_at, dst_ref=dst_at,
                send_sem=send_sems.at[sem_idx],
                recv_sem=recv_sems.at[sem_idx],
                device_id=(target,), device_id_type=pl.DeviceIdType.MESH,
            )

        for b in range(B):
            for h in range(Hl // 2):
                attend(b, h)
        pl.semaphore_wait(barrier_sem, 2)
        loA = (slice(None), slice(None), pl.ds(0, HALF))
        p1 = [
            copy(comm_ref.at[(0, *loA)], comm_ref.at[(1, *loA)], 0, right),
            copy(comm_ref.at[(0, *loA)], comm_ref.at[(2, *loA)], 1, left),
        ]
        p1[0].start()
        p1[1].start()

        for b in range(B):
            for h in range(Hl // 2, Hl):
                attend(b, h)
        hiA = (slice(None), slice(None), pl.ds(HALF, HALF))
        p1 += [
            copy(comm_ref.at[(0, *hiA)], comm_ref.at[(1, *hiA)], 2, right),
            copy(comm_ref.at[(0, *hiA)], comm_ref.at[(2, *hiA)], 3, left),
        ]
        p1[2].start()
        p1[3].start()
        cp_wo.wait()
        contrib(0, init=True)

        lo0 = (0, slice(None), pl.ds(0, HALF))
        hi0 = (0, slice(None), pl.ds(HALF, HALF))
        lo1 = (1, slice(None), pl.ds(0, HALF))
        hi1 = (1, slice(None), pl.ds(HALF, HALF))
        p1[0].wait_recv()
        p2 = [copy(comm_ref.at[(1, *lo0)], comm_ref.at[(3, *lo0)], 4, right)]
        p2[0].start()
        p1[2].wait_recv()
        p2.append(copy(comm_ref.at[(1, *hi0)], comm_ref.at[(3, *hi0)], 5, right))
        p2[1].start()
        contrib(1)
        p1[1].wait_recv()
        p1[3].wait_recv()
        p2.append(copy(comm_ref.at[(2, *lo1)], comm_ref.at[(3, *lo1)], 6, left))
        p2[2].start()
        p2.append(copy(comm_ref.at[(2, *hi1)], comm_ref.at[(3, *hi1)], 7, left))
        p2[3].start()
        contrib(2)
        p2[0].wait_recv()
        p2[1].wait_recv()
        contrib(3, bs=(0,))
        p2[2].wait_recv()
        p2[3].wait_recv()
        contrib(3, bs=(1,))
        for rdma in p1 + p2:
            rdma.wait_send()

    return pl.pallas_call(
        body,
        out_shape=jax.ShapeDtypeStruct((B, Sq, D_out), jnp.float32),
        in_specs=[pl.BlockSpec(memory_space=pltpu.VMEM)] * 2
        + [pl.BlockSpec(memory_space=pl.ANY)] * 3,
        out_specs=pl.BlockSpec(memory_space=pltpu.VMEM),
        scratch_shapes=[
            pltpu.VMEM((2, B, Hl, Skv, Dh), jnp.float32),
            pltpu.VMEM((N_DEV * HD, D_out), jnp.float32),
            pltpu.VMEM((N_DEV, B, Sq, HD), CDT),
            pltpu.SemaphoreType.DMA((2 * B * Hl,)),
            pltpu.SemaphoreType.DMA,
            pltpu.SemaphoreType.DMA((8,)),
            pltpu.SemaphoreType.DMA((8,)),
        ],
        compiler_params=pltpu.CompilerParams(collective_id=0),
    )(x_b, wq_b, K_ext, V_ext, Wo)


# device time: 14730 ns/iter; 1.2222x vs baseline; 1.2222x over previous
import os

import jax
import jax.numpy as jnp
from jax import lax
from jax.experimental import pallas as pl
from jax.experimental.pallas import tpu as pltpu

ABLATE = os.environ.get("ABLATE", "")

N_DEV = 4
WINDOW = 128
CDT = jnp.bfloat16


def kernel(x, Wq, K_ext, V_ext, Wo):
    B, Sq, D_in = x.shape
    _, Skv, Hl, Dh = K_ext.shape
    D_out = Wo.shape[1]
    HD = Hl * Dh
    HALF = HD // 2

    pos = lax.axis_index("i")
    x_b = x.astype(CDT)
    wq_b = lax.dynamic_slice(Wq, (0, pos * HD), (D_in, HD)).astype(CDT)
    k_b = K_ext.reshape(B, Skv, HD).astype(CDT)
    v_b = V_ext.reshape(B, Skv, HD).astype(CDT)
    wo_b = Wo.astype(CDT)

    def body(x_ref, wq_ref, k_ref, v_ref, wo_ref, out_ref,
             comm_ref, send_sems, recv_sems):
        my_pos = lax.axis_index("i")
        right = lax.rem(my_pos + 1, N_DEV)
        left = lax.rem(my_pos + N_DEV - 1, N_DEV)

        barrier_sem = pltpu.get_barrier_semaphore()
        for nbr in (left, right):
            pl.semaphore_signal(
                barrier_sem, inc=1,
                device_id=(nbr,), device_id_type=pl.DeviceIdType.MESH,
            )

        rows = lax.broadcasted_iota(jnp.int32, (Sq, Skv), 0)
        cols = lax.broadcasted_iota(jnp.int32, (Sq, Skv), 1)
        mask = jnp.abs(rows - cols) <= WINDOW

        x_all = x_ref[...].reshape(B * Sq, D_in)
        q_all = jnp.dot(x_all, wq_ref[...],
                        preferred_element_type=jnp.float32) * 0.125

        def attend(b, h):
            qh = q_all[b * Sq:(b + 1) * Sq, h * Dh:(h + 1) * Dh].astype(CDT)
            if ABLATE == "noattn":
                comm_ref[0, b, :, h * Dh:(h + 1) * Dh] = qh
                return
            kh = k_ref[b, :, h * Dh:(h + 1) * Dh]
            vh = v_ref[b, :, h * Dh:(h + 1) * Dh]
            s = lax.dot_general(
                qh, kh, (((1,), (1,)), ((), ())),
                preferred_element_type=jnp.float32,
            )
            w = jnp.exp(jnp.where(mask, s, -1e9))
            recip = 1.0 / jnp.sum(w, axis=1, keepdims=True)
            ctx = jnp.dot(w.astype(CDT), vh,
                          preferred_element_type=jnp.float32)
            comm_ref[0, b, :, h * Dh:(h + 1) * Dh] = (ctx * recip).astype(CDT)

        SLOT_ORIGIN_DELTA = {0: 0, 1: N_DEV - 1, 2: 1, 3: 2}

        def contrib(slot, bs=None, init=False):
            if ABLATE == "nocontrib" and slot > 0:
                return
            origin = lax.rem(my_pos + SLOT_ORIGIN_DELTA[slot], N_DEV)
            wo_block = wo_ref[pl.ds(origin * HD, HD), :]
            if bs is None:
                delta = jnp.dot(
                    comm_ref[slot].reshape(B * Sq, HD), wo_block,
                    preferred_element_type=jnp.float32,
                ).reshape(B, Sq, D_out)
                out_ref[...] = delta if init else out_ref[...] + delta
            else:
                for b in bs:
                    delta = jnp.dot(
                        comm_ref[slot, b], wo_block,
                        preferred_element_type=jnp.float32,
                    )
                    out_ref[b] = delta if init else out_ref[b] + delta

        def copy(src_at, dst_at, sem_idx, target):
            return pltpu.make_async_remote_copy(
                src_ref=src_at, dst_ref=dst_at,
                send_sem=send_sems.at[sem_idx],
                recv_sem=recv_sems.at[sem_idx],
                device_id=(target,), device_id_type=pl.DeviceIdType.MESH,
            )

        for b in range(B):
            for h in range(Hl // 2):
                attend(b, h)
        pl.semaphore_wait(barrier_sem, 2)
        loA = (slice(None), slice(None), pl.ds(0, HALF))
        p1 = [
            copy(comm_ref.at[(0, *loA)], comm_ref.at[(1, *loA)], 0, right),
            copy(comm_ref.at[(0, *loA)], comm_ref.at[(2, *loA)], 1, left),
        ]
        p1[0].start()
        p1[1].start()

        for b in range(B):
            for h in range(Hl // 2, Hl):
                attend(b, h)
        hiA = (slice(None), slice(None), pl.ds(HALF, HALF))
        p1 += [
            copy(comm_ref.at[(0, *hiA)], comm_ref.at[(1, *hiA)], 2, right),
            copy(comm_ref.at[(0, *hiA)], comm_ref.at[(2, *hiA)], 3, left),
        ]
        p1[2].start()
        p1[3].start()
        contrib(0, init=True)

        lo0 = (0, slice(None), pl.ds(0, HALF))
        hi0 = (0, slice(None), pl.ds(HALF, HALF))
        lo1 = (1, slice(None), pl.ds(0, HALF))
        hi1 = (1, slice(None), pl.ds(HALF, HALF))
        p1[0].wait_recv()
        p2 = [copy(comm_ref.at[(1, *lo0)], comm_ref.at[(3, *lo0)], 4, right)]
        p2[0].start()
        p1[2].wait_recv()
        p2.append(copy(comm_ref.at[(1, *hi0)], comm_ref.at[(3, *hi0)], 5, right))
        p2[1].start()
        contrib(1)
        p1[1].wait_recv()
        p1[3].wait_recv()
        p2.append(copy(comm_ref.at[(2, *lo1)], comm_ref.at[(3, *lo1)], 6, left))
        p2[2].start()
        p2.append(copy(comm_ref.at[(2, *hi1)], comm_ref.at[(3, *hi1)], 7, left))
        p2[3].start()
        contrib(2)
        p2[0].wait_recv()
        p2[1].wait_recv()
        contrib(3, bs=(0,))
        p2[2].wait_recv()
        p2[3].wait_recv()
        contrib(3, bs=(1,))
        for rdma in p1 + p2:
            rdma.wait_send()

    return pl.pallas_call(
        body,
        out_shape=jax.ShapeDtypeStruct((B, Sq, D_out), jnp.float32),
        in_specs=[pl.BlockSpec(memory_space=pltpu.VMEM)] * 5,
        out_specs=pl.BlockSpec(memory_space=pltpu.VMEM),
        scratch_shapes=[
            pltpu.VMEM((N_DEV, B, Sq, HD), CDT),
            pltpu.SemaphoreType.DMA((8,)),
            pltpu.SemaphoreType.DMA((8,)),
        ],
        compiler_params=pltpu.CompilerParams(collective_id=0),
    )(x_b, wq_b, k_b, v_b, wo_b)


# device time: 13955 ns/iter; 1.2901x vs baseline; 1.0555x over previous
import os

import jax
import jax.numpy as jnp
from jax import lax
from jax.experimental import pallas as pl
from jax.experimental.pallas import tpu as pltpu

ABLATE = os.environ.get("ABLATE", "")

N_DEV = 4
WINDOW = 128
CDT = jnp.bfloat16


def kernel(x, Wq, K_ext, V_ext, Wo):
    B, Sq, D_in = x.shape
    _, Skv, Hl, Dh = K_ext.shape
    D_out = Wo.shape[1]
    HD = Hl * Dh
    HALF = HD // 2

    pos = lax.axis_index("i")
    x_b = x.astype(CDT)
    wq_b = lax.dynamic_slice(Wq, (0, pos * HD), (D_in, HD)).astype(CDT)
    k_b = K_ext.reshape(B, Skv, HD).astype(CDT)
    v_b = V_ext.reshape(B, Skv, HD).astype(CDT)
    wo_b = Wo.astype(CDT)

    def body(x_ref, wq_ref, k_ref, v_ref, wo_ref, out_ref,
             comm_ref, send_sems, recv_sems):
        my_pos = lax.axis_index("i")
        right = lax.rem(my_pos + 1, N_DEV)
        left = lax.rem(my_pos + N_DEV - 1, N_DEV)

        barrier_sem = pltpu.get_barrier_semaphore()
        for nbr in (left, right):
            pl.semaphore_signal(
                barrier_sem, inc=1,
                device_id=(nbr,), device_id_type=pl.DeviceIdType.MESH,
            )

        rows = lax.broadcasted_iota(jnp.int32, (Sq, Skv), 0)
        cols = lax.broadcasted_iota(jnp.int32, (Sq, Skv), 1)
        mask = jnp.abs(rows - cols) <= WINDOW

        x_all = x_ref[...].reshape(B * Sq, D_in)
        q_all = jnp.dot(x_all, wq_ref[...],
                        preferred_element_type=jnp.float32) * 0.125

        def attend(b, h):
            qh = q_all[b * Sq:(b + 1) * Sq, h * Dh:(h + 1) * Dh].astype(CDT)
            if ABLATE == "noattn":
                comm_ref[0, b, :, h * Dh:(h + 1) * Dh] = qh
                return
            kh = k_ref[b, :, h * Dh:(h + 1) * Dh]
            vh = v_ref[b, :, h * Dh:(h + 1) * Dh]
            s = lax.dot_general(
                qh, kh, (((1,), (1,)), ((), ())),
                preferred_element_type=jnp.float32,
            )
            w = jnp.exp(jnp.where(mask, s, -1e9))
            recip = 1.0 / jnp.sum(w, axis=1, keepdims=True)
            ctx = jnp.dot(w.astype(CDT), vh,
                          preferred_element_type=jnp.float32)
            comm_ref[0, b, :, h * Dh:(h + 1) * Dh] = (ctx * recip).astype(CDT)

        SLOT_ORIGIN_DELTA = {0: 0, 1: N_DEV - 1, 2: 1, 3: 2}

        def contrib(slot, bs=None, init=False):
            if ABLATE == "nocontrib" and slot > 0:
                return
            origin = lax.rem(my_pos + SLOT_ORIGIN_DELTA[slot], N_DEV)
            wo_block = wo_ref[pl.ds(origin * HD, HD), :]
            if bs is None:
                delta = jnp.dot(
                    comm_ref[slot].reshape(B * Sq, HD), wo_block,
                    preferred_element_type=jnp.float32,
                ).reshape(B, Sq, D_out)
                out_ref[...] = delta if init else out_ref[...] + delta
            else:
                for b in bs:
                    delta = jnp.dot(
                        comm_ref[slot, b], wo_block,
                        preferred_element_type=jnp.float32,
                    )
                    out_ref[b] = delta if init else out_ref[b] + delta

        def copy(src_at, dst_at, sem_idx, target):
            return pltpu.make_async_remote_copy(
                src_ref=src_at, dst_ref=dst_at,
                send_sem=send_sems.at[sem_idx],
                recv_sem=recv_sems.at[sem_idx],
                device_id=(target,), device_id_type=pl.DeviceIdType.MESH,
            )

        lanes = {0: pl.ds(0, HALF), 1: pl.ds(HALF, HALF)}
        p1 = []

        def p1_send(b, half):
            q = (b, slice(None), lanes[half])
            for dst_slot, sem_off, tgt in ((1, 0, right), (2, 1, left)):
                r = copy(comm_ref.at[(0, *q)], comm_ref.at[(dst_slot, *q)],
                         4 * half + 2 * b + sem_off, tgt)
                r.start()
                p1.append(r)

        for half in range(2):
            for b in range(B):
                for h in range(half * Hl // 2, (half + 1) * Hl // 2):
                    attend(b, h)
                if half == 0 and b == 0:
                    pl.semaphore_wait(barrier_sem, 2)
                p1_send(b, half)
        contrib(0, init=True)

        p2 = []
        p1[0].wait_recv()
        q = (0, slice(None), lanes[0])
        p2.append(copy(comm_ref.at[(1, *q)], comm_ref.at[(3, *q)], 8, right))
        p2[-1].start()
        p1[4].wait_recv()
        q = (0, slice(None), lanes[1])
        p2.append(copy(comm_ref.at[(1, *q)], comm_ref.at[(3, *q)], 9, right))
        p2[-1].start()
        p1[3].wait_recv()
        q = (1, slice(None), lanes[0])
        p2.append(copy(comm_ref.at[(2, *q)], comm_ref.at[(3, *q)], 10, left))
        p2[-1].start()
        p1[7].wait_recv()
        q = (1, slice(None), lanes[1])
        p2.append(copy(comm_ref.at[(2, *q)], comm_ref.at[(3, *q)], 11, left))
        p2[-1].start()

        p1[2].wait_recv()
        p1[6].wait_recv()
        contrib(1)
        p1[1].wait_recv()
        p1[5].wait_recv()
        contrib(2)
        p2[0].wait_recv()
        p2[1].wait_recv()
        contrib(3, bs=(0,))
        p2[2].wait_recv()
        p2[3].wait_recv()
        contrib(3, bs=(1,))
        for rdma in p1 + p2:
            rdma.wait_send()

    return pl.pallas_call(
        body,
        out_shape=jax.ShapeDtypeStruct((B, Sq, D_out), jnp.float32),
        in_specs=[pl.BlockSpec(memory_space=pltpu.VMEM)] * 5,
        out_specs=pl.BlockSpec(memory_space=pltpu.VMEM),
        scratch_shapes=[
            pltpu.VMEM((N_DEV, B, Sq, HD), CDT),
            pltpu.SemaphoreType.DMA((12,)),
            pltpu.SemaphoreType.DMA((12,)),
        ],
        compiler_params=pltpu.CompilerParams(collective_id=0),
    )(x_b, wq_b, k_b, v_b, wo_b)
